# baseline (speedup 1.0000x reference)
import jax
import jax.numpy as jnp
from jax.experimental import pallas as pl
from jax.experimental.pallas import tpu as pltpu

_LIMIT = float(jnp.pi) / 2.0
_WEIGHT = 0.01

_N = 16384
_D = 72
_BANDS = (1, 3)
_BAND_ROWS = ((1, 2), (0,))


_GRID = 8
_CH = _N // _GRID


def _tc_body(a_ref, b_ref, o_ref, acc_ref):
    i = pl.program_id(0)

    @pl.when(i == 0)
    def _():
        acc_ref[0] = 0.0

    acc = jnp.float32(0.0)
    for ref, rows in zip((a_ref, b_ref), _BAND_ROWS):
        v = ref[...]
        r = jax.lax.broadcasted_iota(jnp.int32, v.shape, 0)
        keep = r == rows[0]
        for extra in rows[1:]:
            keep = keep | (r == extra)
        keep = keep & (jnp.abs(v) > _LIMIT)
        p = jnp.where(keep, v, 0.0)
        acc = acc + jnp.sum(p * p)
    acc_ref[0] += acc

    @pl.when(i == _GRID - 1)
    def _():
        o_ref[0] = acc_ref[0] * _WEIGHT


@jax.jit
def kernel(pose):
    xt = pose.T
    out = pl.pallas_call(
        _tc_body,
        grid=(_GRID,),
        in_specs=[
            pl.BlockSpec((8, _CH), lambda i, b=b: (b, i)) for b in _BANDS
        ],
        out_specs=pl.BlockSpec(memory_space=pltpu.SMEM),
        out_shape=jax.ShapeDtypeStruct((1,), jnp.float32),
        scratch_shapes=[pltpu.SMEM((1,), jnp.float32)],
        compiler_params=pltpu.CompilerParams(
            dimension_semantics=("arbitrary",),
        ),
    )(xt, xt)
    return out[0]
